# baseline (speedup 1.0000x reference)
import functools

import numpy as np

import jax
import jax.numpy as jnp
from jax import lax
from jax.experimental import pallas as pl
from jax.experimental.pallas import tpu as pltpu
from jax.experimental.pallas import tpu_sc as plsc

MAXZ = 84
NRP = 24
NATOMS = 1048576

NC = 2
NS = 16
NW = NC * NS
L = 16

PER_W = NATOMS // NW
CHUNK = 2048
NCHUNK = PER_W // CHUNK
BPC = CHUNK // L
VPB = L * NRP // L


def _gather_sc(Z, p_flat):
    mesh = plsc.VectorSubcoreMesh(core_axis_name="c", subcore_axis_name="s")

    @functools.partial(
        pl.kernel,
        mesh=mesh,
        out_type=jax.ShapeDtypeStruct((NATOMS * NRP,), jnp.float32),
        scratch_types=[
            pltpu.VMEM((MAXZ * NRP,), jnp.float32),
            pltpu.VMEM((2, CHUNK + L), jnp.int32),
            pltpu.VMEM((2, CHUNK * NRP), jnp.float32),
            pltpu.SemaphoreType.DMA((2,)),
            pltpu.SemaphoreType.DMA((2,)),
            pltpu.SemaphoreType.DMA,
        ],
        compiler_params=pltpu.CompilerParams(
            use_tc_tiling_on_sc=False, needs_layout_passes=False
        ),
    )
    def k(z_hbm, p_hbm, out_hbm, table_v, idx_v, rows_v, isem, osem, tsem):
        wid = lax.axis_index("s") * NC + lax.axis_index("c")
        base = wid * PER_W
        pltpu.async_copy(p_hbm, table_v, tsem).wait()

        lane = lax.iota(jnp.int32, L)
        half = (lane >= 8).astype(jnp.int32)
        apat = [lane * 0, half, lane * 0 + 1]
        jpat = [lane, lane + 16 - 24 * half, lane + 8]

        _dn = lax.GatherDimensionNumbers(
            offset_dims=(), collapsed_slice_dims=(0,), start_index_map=(0,)
        )

        def vperm(x, idx):
            return lax.gather(
                x, idx[:, None], _dn, (1,),
                mode=lax.GatherScatterMode.PROMISE_IN_BOUNDS,
            )

        idx_cp = [None, None]
        out_cp = [None, None]

        def start_idx(c):
            s = c % 2
            idx_cp[s] = pltpu.async_copy(
                z_hbm.at[pl.ds(base + c * CHUNK, CHUNK)],
                idx_v.at[s].at[pl.ds(0, CHUNK)],
                isem.at[s],
            )

        def start_write(c):
            s = c % 2
            out_cp[s] = pltpu.async_copy(
                rows_v.at[s],
                out_hbm.at[pl.ds((base + c * CHUNK) * NRP, CHUNK * NRP)],
                osem.at[s],
            )

        def compute(c):
            s = c % 2
            zref = idx_v.at[s]
            rref = rows_v.at[s]

            def body(i, carry):
                obase = i * (L * NRP)
                for m in range(L // 2):
                    zs = zref[pl.ds(i * L + 2 * m, L)]
                    z24 = zs * NRP
                    for r in range(3):
                        zsel = vperm(z24, apat[r])
                        g = plsc.load_gather(table_v, [zsel + jpat[r]])
                        rref[pl.ds(obase + (3 * m + r) * L, L)] = g
                return carry

            lax.fori_loop(0, BPC, body, 0, unroll=2)

        start_idx(0)
        start_idx(1)

        for c in range(NCHUNK):
            s = c % 2
            idx_cp[s].wait()
            if c >= 2:
                out_cp[s].wait()
            compute(c)
            start_write(c)
            if c + 2 < NCHUNK:
                start_idx(c + 2)

        out_cp[0].wait()
        out_cp[1].wait()

    return k(Z, p_flat)


def kernel(Z, p, alpha, chi):
    Z32 = Z.astype(jnp.int32)
    out_flat = _gather_sc(Z32, p.reshape(-1))
    return (out_flat.reshape(NATOMS, NRP), alpha, chi)


# speedup vs baseline: 1.2991x; 1.2991x over previous
import functools

import jax
import jax.numpy as jnp
from jax import lax
from jax.experimental import pallas as pl
from jax.experimental.pallas import tpu as pltpu
from jax.experimental.pallas import tpu_sc as plsc

MAXZ = 84
NRP = 24
NATOMS = 1048576

NC = 2
NS = 16
NW = NC * NS
L = 16

PER_W = NATOMS // NW
CHUNK = 512
NCHUNK = PER_W // CHUNK


def _gather_sc(Z, p_flat):
    mesh = plsc.VectorSubcoreMesh(core_axis_name="c", subcore_axis_name="s")

    @functools.partial(
        pl.kernel,
        mesh=mesh,
        out_type=jax.ShapeDtypeStruct((NATOMS * NRP,), jnp.float32),
        scratch_types=[
            pltpu.VMEM((MAXZ * NRP,), jnp.float32),
            pltpu.VMEM((2, CHUNK), jnp.int32),
            pltpu.VMEM((2, CHUNK * NRP), jnp.float32),
            pltpu.SemaphoreType.DMA((2,)),
            pltpu.SemaphoreType.DMA((2,)),
            pltpu.SemaphoreType.DMA,
        ],
        compiler_params=pltpu.CompilerParams(
            use_tc_tiling_on_sc=False, needs_layout_passes=False
        ),
    )
    def k(z_hbm, p_hbm, out_hbm, table_v, zs_v, rows_v, isem, osem, tsem):
        wid = lax.axis_index("s") * NC + lax.axis_index("c")
        base = wid * PER_W
        pltpu.async_copy(p_hbm, table_v, tsem).wait()

        def start_idx(c, s):
            pltpu.async_copy(
                z_hbm.at[pl.ds(base + c * CHUNK, CHUNK)], zs_v.at[s], isem.at[s]
            )

        def wait_idx(s):
            pltpu.make_async_copy(
                z_hbm.at[pl.ds(base, CHUNK)], zs_v.at[s], isem.at[s]
            ).wait()

        def start_write(c, s):
            pltpu.async_copy(
                rows_v.at[s],
                out_hbm.at[pl.ds((base + c * CHUNK) * NRP, CHUNK * NRP)],
                osem.at[s],
            )

        def wait_write(s):
            pltpu.make_async_copy(
                rows_v.at[s],
                out_hbm.at[pl.ds(base * NRP, CHUNK * NRP)],
                osem.at[s],
            ).wait()

        def compute(s):
            rref = rows_v.at[s]
            zref = zs_v.at[s]

            def body(v, carry):
                zvec = zref[pl.ds(v * L, L)] * NRP
                gs = []
                for l in range(L):
                    zoff = zvec[l]
                    gs.append((table_v[pl.ds(zoff, L)],
                               table_v[pl.ds(zoff + NRP - L, L)]))
                for l, (g0, g1) in enumerate(gs):
                    abase = (v * L + l) * NRP
                    rref[pl.ds(abase, L)] = g0
                    rref[pl.ds(abase + NRP - L, L)] = g1
                return carry

            lax.fori_loop(0, CHUNK // L, body, 0, unroll=1)

        start_idx(0, 0)
        start_idx(1, 1)
        wait_idx(0)
        compute(0)
        start_write(0, 0)
        start_idx(2, 0)
        wait_idx(1)
        compute(1)
        start_write(1, 1)
        start_idx(3, 1)

        def pair(cp, carry):
            c0 = cp * 2
            for s in range(2):
                c = c0 + s
                wait_idx(s)
                wait_write(s)
                compute(s)
                start_write(c, s)

                @pl.when(c + 2 < NCHUNK)
                def _():
                    start_idx(c + 2, s)
            return carry

        lax.fori_loop(1, NCHUNK // 2, pair, 0)

        wait_write(0)
        wait_write(1)

    return k(Z, p_flat)


def kernel(Z, p, alpha, chi):
    Z32 = Z.astype(jnp.int32)
    out_flat = _gather_sc(Z32, p.reshape(-1))
    return (out_flat.reshape(NATOMS, NRP), alpha, chi)
